# initial kernel scaffold (unmeasured)
import jax
import jax.numpy as jnp
from jax import lax
from jax.experimental import pallas as pl
from jax.experimental.pallas import tpu as pltpu

N_DEV = 32


def kernel(x, W1, W2):
    m, d = x.shape
    n = W2.shape[1]
    n_steps = N_DEV - 1

    def body(x_ref, w1_ref, w2_ref, out_ref, xall, sbuf, rbuf,
             ag_send, ag_recv, rs_send, rs_recv, credit):
        me = lax.axis_index("i")
        left = lax.rem(me + N_DEV - 1, N_DEV)
        right = lax.rem(me + 1, N_DEV)

        barrier = pltpu.get_barrier_semaphore()
        pl.semaphore_signal(barrier, inc=1, device_id=(left,),
                            device_id_type=pl.DeviceIdType.MESH)
        pl.semaphore_signal(barrier, inc=1, device_id=(right,),
                            device_id_type=pl.DeviceIdType.MESH)
        pl.semaphore_wait(barrier, 2)

        xall[0] = x_ref[...]

        def ag_step(s, carry):
            rdma = pltpu.make_async_remote_copy(
                src_ref=xall.at[s],
                dst_ref=xall.at[s + 1],
                send_sem=ag_send.at[s],
                recv_sem=ag_recv.at[s],
                device_id=(right,),
                device_id_type=pl.DeviceIdType.MESH,
            )
            rdma.start()
            rdma.wait()
            return carry

        lax.fori_loop(0, n_steps, ag_step, 0)

        w1 = w1_ref[...]
        w2 = w2_ref[...]

        def partial_for(xb):
            h = jnp.dot(xb, w1, preferred_element_type=jnp.float32)
            h = h * jax.nn.sigmoid(h)
            return jnp.dot(h, w2, preferred_element_type=jnp.float32)

        def rs_step(t, carry):
            slot = lax.rem(t, 2)
            prev_slot = lax.rem(t + 1, 2)
            xb = xall[pl.ds(t + 1, 1)].reshape(m, d)
            p = partial_for(xb)
            prev = rbuf[pl.ds(prev_slot, 1)].reshape(m, n)
            acc = jnp.where(t > 0, p + prev, p)
            sbuf[pl.ds(slot, 1)] = acc.reshape(1, m, n)

            @pl.when(jnp.logical_and(t >= 1, t <= n_steps - 2))
            def _():
                pl.semaphore_signal(credit, inc=1, device_id=(left,),
                                    device_id_type=pl.DeviceIdType.MESH)

            @pl.when(t >= 2)
            def _():
                pl.semaphore_wait(credit, 1)

            rdma = pltpu.make_async_remote_copy(
                src_ref=sbuf.at[slot],
                dst_ref=rbuf.at[slot],
                send_sem=rs_send.at[t],
                recv_sem=rs_recv.at[t],
                device_id=(right,),
                device_id_type=pl.DeviceIdType.MESH,
            )
            rdma.start()
            rdma.wait()
            return carry

        lax.fori_loop(0, n_steps, rs_step, 0)

        p_own = partial_for(xall[0])
        out_ref[...] = rbuf[0] + p_own

    return pl.pallas_call(
        body,
        out_shape=jax.ShapeDtypeStruct((m, n), jnp.float32),
        in_specs=[pl.BlockSpec(memory_space=pltpu.VMEM)] * 3,
        out_specs=pl.BlockSpec(memory_space=pltpu.VMEM),
        scratch_shapes=[
            pltpu.VMEM((N_DEV, m, d), jnp.float32),
            pltpu.VMEM((2, m, n), jnp.float32),
            pltpu.VMEM((2, m, n), jnp.float32),
            pltpu.SemaphoreType.DMA((n_steps,)),
            pltpu.SemaphoreType.DMA((n_steps,)),
            pltpu.SemaphoreType.DMA((n_steps,)),
            pltpu.SemaphoreType.DMA((n_steps,)),
            pltpu.SemaphoreType.REGULAR,
        ],
        compiler_params=pltpu.CompilerParams(collective_id=0),
    )(x, W1, W2)


# baseline (device time: 912857 ns/iter reference)
import jax
import jax.numpy as jnp
from jax import lax
from jax.experimental import pallas as pl
from jax.experimental.pallas import tpu as pltpu

N_DEV = 32


def kernel(x, W1, W2):
    m, d = x.shape
    n = W2.shape[1]
    n_steps = N_DEV - 1

    def body(x_ref, w1_ref, w2_ref, out_ref, xall, sbuf, rbuf,
             ag_send, ag_recv, rs_send, rs_recv, credit):
        me = lax.axis_index("i")
        left = lax.rem(me + N_DEV - 1, N_DEV)
        right = lax.rem(me + 1, N_DEV)

        barrier = pltpu.get_barrier_semaphore()
        pl.semaphore_signal(barrier, inc=1, device_id=(left,),
                            device_id_type=pl.DeviceIdType.MESH)
        pl.semaphore_signal(barrier, inc=1, device_id=(right,),
                            device_id_type=pl.DeviceIdType.MESH)
        pl.semaphore_wait(barrier, 2)

        xall[0] = x_ref[...]

        def ag_step(s, carry):
            rdma = pltpu.make_async_remote_copy(
                src_ref=xall.at[s],
                dst_ref=xall.at[s + 1],
                send_sem=ag_send.at[s],
                recv_sem=ag_recv.at[s],
                device_id=(right,),
                device_id_type=pl.DeviceIdType.MESH,
            )
            rdma.start()
            rdma.wait()
            return carry

        lax.fori_loop(0, n_steps, ag_step, 0)

        w1 = w1_ref[...]
        w2 = w2_ref[...]

        def partial_for(xb):
            h = jnp.dot(xb, w1, preferred_element_type=jnp.float32)
            h = h * jax.nn.sigmoid(h)
            return jnp.dot(h, w2, preferred_element_type=jnp.float32)

        def rs_step(t, carry):
            slot = lax.rem(t, 2)
            prev_slot = lax.rem(t + 1, 2)
            xb = xall[pl.ds(t + 1, 1)].reshape(m, d)
            p = partial_for(xb)
            prev = rbuf[pl.ds(prev_slot, 1)].reshape(m, n)
            acc = jnp.where(t > 0, p + prev, p)
            sbuf[pl.ds(slot, 1)] = acc.reshape(1, m, n)

            @pl.when(jnp.logical_and(t >= 1, t <= n_steps - 2))
            def _():
                pl.semaphore_signal(credit, inc=1, device_id=(left,),
                                    device_id_type=pl.DeviceIdType.MESH)

            @pl.when(t >= 2)
            def _():
                pl.semaphore_wait(credit, 1)

            rdma = pltpu.make_async_remote_copy(
                src_ref=sbuf.at[slot],
                dst_ref=rbuf.at[slot],
                send_sem=rs_send.at[t],
                recv_sem=rs_recv.at[t],
                device_id=(right,),
                device_id_type=pl.DeviceIdType.MESH,
            )
            rdma.start()
            rdma.wait()
            return carry

        lax.fori_loop(0, n_steps, rs_step, 0)

        p_own = partial_for(xall[0])
        out_ref[...] = rbuf[0] + p_own

    return pl.pallas_call(
        body,
        out_shape=jax.ShapeDtypeStruct((m, n), jnp.float32),
        in_specs=[pl.BlockSpec(memory_space=pltpu.VMEM)] * 3,
        out_specs=pl.BlockSpec(memory_space=pltpu.VMEM),
        scratch_shapes=[
            pltpu.VMEM((N_DEV, m, d), jnp.float32),
            pltpu.VMEM((2, m, n), jnp.float32),
            pltpu.VMEM((2, m, n), jnp.float32),
            pltpu.SemaphoreType.DMA((n_steps,)),
            pltpu.SemaphoreType.DMA((n_steps,)),
            pltpu.SemaphoreType.DMA((n_steps,)),
            pltpu.SemaphoreType.DMA((n_steps,)),
            pltpu.SemaphoreType.REGULAR,
        ],
        compiler_params=pltpu.CompilerParams(
            collective_id=0,
            vmem_limit_bytes=100 * 1024 * 1024,
        ),
    )(x, W1, W2)


# device time: 727778 ns/iter; 1.2543x vs baseline; 1.2543x over previous
import jax
import jax.numpy as jnp
from jax import lax
from jax.experimental import pallas as pl
from jax.experimental.pallas import tpu as pltpu

N_DEV = 32


def kernel(x, W1, W2):
    m, d = x.shape
    n = W2.shape[1]
    mh = m // 2
    n_steps = N_DEV - 1

    def body(x_ref, w1_ref, w2_ref, out_ref,
             xallA, xallB, sbufA, rbufA, sbufB, rbufB,
             agA_send, agA_recv, agB_send, agB_recv,
             rsA_send, rsA_recv, rsB_send, rsB_recv,
             creditA, creditB):
        me = lax.axis_index("i")
        left = lax.rem(me + N_DEV - 1, N_DEV)
        right = lax.rem(me + 1, N_DEV)

        def agA(s):
            return pltpu.make_async_remote_copy(
                src_ref=xallA.at[s], dst_ref=xallA.at[s + 1],
                send_sem=agA_send.at[s], recv_sem=agA_recv.at[s],
                device_id=(right,), device_id_type=pl.DeviceIdType.MESH)

        def agB(s):
            return pltpu.make_async_remote_copy(
                src_ref=xallB.at[s], dst_ref=xallB.at[s + 1],
                send_sem=agB_send.at[s], recv_sem=agB_recv.at[s],
                device_id=(left,), device_id_type=pl.DeviceIdType.MESH)

        def rsA(t):
            slot = lax.rem(t, 2)
            return pltpu.make_async_remote_copy(
                src_ref=sbufA.at[slot], dst_ref=rbufA.at[slot],
                send_sem=rsA_send.at[t], recv_sem=rsA_recv.at[t],
                device_id=(right,), device_id_type=pl.DeviceIdType.MESH)

        def rsB(t):
            slot = lax.rem(t, 2)
            return pltpu.make_async_remote_copy(
                src_ref=sbufB.at[slot], dst_ref=rbufB.at[slot],
                send_sem=rsB_send.at[t], recv_sem=rsB_recv.at[t],
                device_id=(left,), device_id_type=pl.DeviceIdType.MESH)

        barrier = pltpu.get_barrier_semaphore()
        pl.semaphore_signal(barrier, inc=1, device_id=(left,),
                            device_id_type=pl.DeviceIdType.MESH)
        pl.semaphore_signal(barrier, inc=1, device_id=(right,),
                            device_id_type=pl.DeviceIdType.MESH)
        pl.semaphore_wait(barrier, 2)

        xallA[0] = x_ref[:mh, :]
        xallB[0] = x_ref[mh:, :]

        agA(0).start()
        agB(0).start()

        w1 = w1_ref[...]
        w2 = w2_ref[...]

        def partial_for(xb):
            h = jnp.dot(xb, w1, preferred_element_type=jnp.float32)
            h = h * jax.nn.sigmoid(h)
            return jnp.dot(h, w2, preferred_element_type=jnp.float32)

        def step(t, carry):
            slot = lax.rem(t, 2)
            prev_slot = lax.rem(t + 1, 2)

            agA(t).wait_recv()
            agB(t).wait_recv()

            @pl.when(t <= n_steps - 2)
            def _():
                agA(t + 1).start()
                agB(t + 1).start()

            agA(t).wait_send()
            agB(t).wait_send()

            pA = partial_for(xallA[pl.ds(t + 1, 1)].reshape(mh, d))
            pB = partial_for(xallB[pl.ds(t + 1, 1)].reshape(mh, d))

            @pl.when(t > 0)
            def _():
                rsA(t - 1).wait_recv()
                rsB(t - 1).wait_recv()

            prevA = rbufA[pl.ds(prev_slot, 1)].reshape(mh, n)
            prevB = rbufB[pl.ds(prev_slot, 1)].reshape(mh, n)
            accA = jnp.where(t > 0, pA + prevA, pA)
            accB = jnp.where(t > 0, pB + prevB, pB)

            @pl.when(jnp.logical_and(t >= 1, t <= n_steps - 2))
            def _():
                pl.semaphore_signal(creditA, inc=1, device_id=(left,),
                                    device_id_type=pl.DeviceIdType.MESH)
                pl.semaphore_signal(creditB, inc=1, device_id=(right,),
                                    device_id_type=pl.DeviceIdType.MESH)

            @pl.when(t >= 2)
            def _():
                rsA(t - 2).wait_send()
                rsB(t - 2).wait_send()

            sbufA[pl.ds(slot, 1)] = accA.reshape(1, mh, n)
            sbufB[pl.ds(slot, 1)] = accB.reshape(1, mh, n)

            @pl.when(t >= 2)
            def _():
                pl.semaphore_wait(creditA, 1)
                pl.semaphore_wait(creditB, 1)

            rsA(t).start()
            rsB(t).start()
            return carry

        lax.fori_loop(0, n_steps, step, 0)

        rsA(n_steps - 2).wait_send()
        rsB(n_steps - 2).wait_send()
        rsA(n_steps - 1).wait_send()
        rsB(n_steps - 1).wait_send()
        rsA(n_steps - 1).wait_recv()
        rsB(n_steps - 1).wait_recv()

        last_slot = (n_steps - 1) % 2
        out_ref[:mh, :] = rbufA[last_slot] + partial_for(xallA[0])
        out_ref[mh:, :] = rbufB[last_slot] + partial_for(xallB[0])

    return pl.pallas_call(
        body,
        out_shape=jax.ShapeDtypeStruct((m, n), jnp.float32),
        in_specs=[pl.BlockSpec(memory_space=pltpu.VMEM)] * 3,
        out_specs=pl.BlockSpec(memory_space=pltpu.VMEM),
        scratch_shapes=[
            pltpu.VMEM((N_DEV, mh, d), jnp.float32),
            pltpu.VMEM((N_DEV, mh, d), jnp.float32),
            pltpu.VMEM((2, mh, n), jnp.float32),
            pltpu.VMEM((2, mh, n), jnp.float32),
            pltpu.VMEM((2, mh, n), jnp.float32),
            pltpu.VMEM((2, mh, n), jnp.float32),
            pltpu.SemaphoreType.DMA((n_steps,)),
            pltpu.SemaphoreType.DMA((n_steps,)),
            pltpu.SemaphoreType.DMA((n_steps,)),
            pltpu.SemaphoreType.DMA((n_steps,)),
            pltpu.SemaphoreType.DMA((n_steps,)),
            pltpu.SemaphoreType.DMA((n_steps,)),
            pltpu.SemaphoreType.DMA((n_steps,)),
            pltpu.SemaphoreType.DMA((n_steps,)),
            pltpu.SemaphoreType.REGULAR,
            pltpu.SemaphoreType.REGULAR,
        ],
        compiler_params=pltpu.CompilerParams(
            collective_id=0,
            vmem_limit_bytes=100 * 1024 * 1024,
        ),
    )(x, W1, W2)


# device time: 374608 ns/iter; 2.4368x vs baseline; 1.9428x over previous
import jax
import jax.numpy as jnp
import numpy as np
from jax import lax
from jax.experimental import pallas as pl
from jax.experimental.pallas import tpu as pltpu

N_DEV = 32


def _ring_tables():
    path_yz = [(0, 0), (1, 0), (2, 0), (3, 0),
               (3, 1), (2, 1), (1, 1), (0, 1),
               (0, 2), (1, 2), (2, 2), (3, 2),
               (3, 3), (2, 3), (1, 3), (0, 3)]
    cycle = [(0, y, z) for (y, z) in path_yz] + \
            [(1, y, z) for (y, z) in reversed(path_yz)]

    def lid(x, y, z):
        return z * 8 + y * 2 + (x if y % 2 == 0 else 1 - x)

    ring = [lid(*c) for c in cycle]
    right = [0] * N_DEV
    left = [0] * N_DEV
    for r, lg in enumerate(ring):
        right[lg] = ring[(r + 1) % N_DEV]
        left[lg] = ring[(r - 1) % N_DEV]
    return np.array(right, np.int32), np.array(left, np.int32)


_RIGHT_TBL, _LEFT_TBL = _ring_tables()


def kernel(x, W1, W2):
    m, d = x.shape
    n = W2.shape[1]
    mh = m // 2
    n_steps = N_DEV - 1

    def body(nbrs_ref, x_ref, w1_ref, w2_ref, out_ref,
             xallA, xallB, sbufA, rbufA, sbufB, rbufB,
             agA_send, agA_recv, agB_send, agB_recv,
             rsA_send, rsA_recv, rsB_send, rsB_recv,
             creditA, creditB):
        left = nbrs_ref[0]
        right = nbrs_ref[1]

        def agA(s):
            return pltpu.make_async_remote_copy(
                src_ref=xallA.at[s], dst_ref=xallA.at[s + 1],
                send_sem=agA_send.at[s], recv_sem=agA_recv.at[s],
                device_id=(right,), device_id_type=pl.DeviceIdType.MESH)

        def agB(s):
            return pltpu.make_async_remote_copy(
                src_ref=xallB.at[s], dst_ref=xallB.at[s + 1],
                send_sem=agB_send.at[s], recv_sem=agB_recv.at[s],
                device_id=(left,), device_id_type=pl.DeviceIdType.MESH)

        def rsA(t):
            slot = lax.rem(t, 2)
            return pltpu.make_async_remote_copy(
                src_ref=sbufA.at[slot], dst_ref=rbufA.at[slot],
                send_sem=rsA_send.at[t], recv_sem=rsA_recv.at[t],
                device_id=(right,), device_id_type=pl.DeviceIdType.MESH)

        def rsB(t):
            slot = lax.rem(t, 2)
            return pltpu.make_async_remote_copy(
                src_ref=sbufB.at[slot], dst_ref=rbufB.at[slot],
                send_sem=rsB_send.at[t], recv_sem=rsB_recv.at[t],
                device_id=(left,), device_id_type=pl.DeviceIdType.MESH)

        barrier = pltpu.get_barrier_semaphore()
        pl.semaphore_signal(barrier, inc=1, device_id=(left,),
                            device_id_type=pl.DeviceIdType.MESH)
        pl.semaphore_signal(barrier, inc=1, device_id=(right,),
                            device_id_type=pl.DeviceIdType.MESH)
        pl.semaphore_wait(barrier, 2)

        xallA[0] = x_ref[:mh, :]
        xallB[0] = x_ref[mh:, :]

        agA(0).start()
        agB(0).start()

        w1 = w1_ref[...]
        w2 = w2_ref[...]

        def partial_for(xb):
            h = jnp.dot(xb, w1, preferred_element_type=jnp.float32)
            h = h * jax.nn.sigmoid(h)
            return jnp.dot(h, w2, preferred_element_type=jnp.float32)

        def step(t, carry):
            slot = lax.rem(t, 2)
            prev_slot = lax.rem(t + 1, 2)

            agA(t).wait_recv()
            agB(t).wait_recv()

            @pl.when(t <= n_steps - 2)
            def _():
                agA(t + 1).start()
                agB(t + 1).start()

            agA(t).wait_send()
            agB(t).wait_send()

            pA = partial_for(xallA[pl.ds(t + 1, 1)].reshape(mh, d))
            pB = partial_for(xallB[pl.ds(t + 1, 1)].reshape(mh, d))

            @pl.when(t > 0)
            def _():
                rsA(t - 1).wait_recv()
                rsB(t - 1).wait_recv()

            prevA = rbufA[pl.ds(prev_slot, 1)].reshape(mh, n)
            prevB = rbufB[pl.ds(prev_slot, 1)].reshape(mh, n)
            accA = jnp.where(t > 0, pA + prevA, pA)
            accB = jnp.where(t > 0, pB + prevB, pB)

            @pl.when(jnp.logical_and(t >= 1, t <= n_steps - 2))
            def _():
                pl.semaphore_signal(creditA, inc=1, device_id=(left,),
                                    device_id_type=pl.DeviceIdType.MESH)
                pl.semaphore_signal(creditB, inc=1, device_id=(right,),
                                    device_id_type=pl.DeviceIdType.MESH)

            @pl.when(t >= 2)
            def _():
                rsA(t - 2).wait_send()
                rsB(t - 2).wait_send()

            sbufA[pl.ds(slot, 1)] = accA.reshape(1, mh, n)
            sbufB[pl.ds(slot, 1)] = accB.reshape(1, mh, n)

            @pl.when(t >= 2)
            def _():
                pl.semaphore_wait(creditA, 1)
                pl.semaphore_wait(creditB, 1)

            rsA(t).start()
            rsB(t).start()
            return carry

        lax.fori_loop(0, n_steps, step, 0)

        rsA(n_steps - 2).wait_send()
        rsB(n_steps - 2).wait_send()
        rsA(n_steps - 1).wait_send()
        rsB(n_steps - 1).wait_send()
        rsA(n_steps - 1).wait_recv()
        rsB(n_steps - 1).wait_recv()

        last_slot = (n_steps - 1) % 2
        out_ref[:mh, :] = rbufA[last_slot] + partial_for(xallA[0])
        out_ref[mh:, :] = rbufB[last_slot] + partial_for(xallB[0])

    me = lax.axis_index("i")
    nbrs = jnp.stack(
        [jnp.asarray(_LEFT_TBL)[me], jnp.asarray(_RIGHT_TBL)[me]]
    ).astype(jnp.int32)

    return pl.pallas_call(
        body,
        out_shape=jax.ShapeDtypeStruct((m, n), jnp.float32),
        in_specs=[pl.BlockSpec(memory_space=pltpu.SMEM)]
        + [pl.BlockSpec(memory_space=pltpu.VMEM)] * 3,
        out_specs=pl.BlockSpec(memory_space=pltpu.VMEM),
        scratch_shapes=[
            pltpu.VMEM((N_DEV, mh, d), jnp.float32),
            pltpu.VMEM((N_DEV, mh, d), jnp.float32),
            pltpu.VMEM((2, mh, n), jnp.float32),
            pltpu.VMEM((2, mh, n), jnp.float32),
            pltpu.VMEM((2, mh, n), jnp.float32),
            pltpu.VMEM((2, mh, n), jnp.float32),
            pltpu.SemaphoreType.DMA((n_steps,)),
            pltpu.SemaphoreType.DMA((n_steps,)),
            pltpu.SemaphoreType.DMA((n_steps,)),
            pltpu.SemaphoreType.DMA((n_steps,)),
            pltpu.SemaphoreType.DMA((n_steps,)),
            pltpu.SemaphoreType.DMA((n_steps,)),
            pltpu.SemaphoreType.DMA((n_steps,)),
            pltpu.SemaphoreType.DMA((n_steps,)),
            pltpu.SemaphoreType.REGULAR,
            pltpu.SemaphoreType.REGULAR,
        ],
        compiler_params=pltpu.CompilerParams(
            collective_id=0,
            vmem_limit_bytes=100 * 1024 * 1024,
        ),
    )(nbrs, x, W1, W2)


# device time: 371910 ns/iter; 2.4545x vs baseline; 1.0073x over previous
import jax
import jax.numpy as jnp
import numpy as np
from jax import lax
from jax.experimental import pallas as pl
from jax.experimental.pallas import tpu as pltpu

N_DEV = 32


def _ring_tables():
    path_yz = [(0, 0), (1, 0), (2, 0), (3, 0),
               (3, 1), (2, 1), (1, 1), (0, 1),
               (0, 2), (1, 2), (2, 2), (3, 2),
               (3, 3), (2, 3), (1, 3), (0, 3)]
    cycle = [(0, y, z) for (y, z) in path_yz] + \
            [(1, y, z) for (y, z) in reversed(path_yz)]

    def lid(x, y, z):
        return z * 8 + y * 2 + (x if y % 2 == 0 else 1 - x)

    ring = [lid(*c) for c in cycle]
    right = [0] * N_DEV
    left = [0] * N_DEV
    for r, lg in enumerate(ring):
        right[lg] = ring[(r + 1) % N_DEV]
        left[lg] = ring[(r - 1) % N_DEV]
    return np.array(right, np.int32), np.array(left, np.int32)


_RIGHT_TBL, _LEFT_TBL = _ring_tables()


def kernel(x, W1, W2):
    m, d = x.shape
    n = W2.shape[1]
    mh = m // 2
    n_steps = N_DEV - 1

    def body(nbrs_ref, x_ref, w1_ref, w2_ref, out_ref,
             xallA, xallB, sbufA, rbufA, sbufB, rbufB,
             pownA, pownB,
             agA_send, agA_recv, agB_send, agB_recv,
             rsA_send, rsA_recv, rsB_send, rsB_recv,
             creditA, creditB):
        left = nbrs_ref[0]
        right = nbrs_ref[1]

        def agA(s):
            return pltpu.make_async_remote_copy(
                src_ref=xallA.at[s], dst_ref=xallA.at[s + 1],
                send_sem=agA_send.at[s], recv_sem=agA_recv.at[s],
                device_id=(right,), device_id_type=pl.DeviceIdType.MESH)

        def agB(s):
            return pltpu.make_async_remote_copy(
                src_ref=xallB.at[s], dst_ref=xallB.at[s + 1],
                send_sem=agB_send.at[s], recv_sem=agB_recv.at[s],
                device_id=(left,), device_id_type=pl.DeviceIdType.MESH)

        def rsA(t):
            slot = lax.rem(t, 2)
            return pltpu.make_async_remote_copy(
                src_ref=sbufA.at[slot], dst_ref=rbufA.at[slot],
                send_sem=rsA_send.at[t], recv_sem=rsA_recv.at[t],
                device_id=(right,), device_id_type=pl.DeviceIdType.MESH)

        def rsB(t):
            slot = lax.rem(t, 2)
            return pltpu.make_async_remote_copy(
                src_ref=sbufB.at[slot], dst_ref=rbufB.at[slot],
                send_sem=rsB_send.at[t], recv_sem=rsB_recv.at[t],
                device_id=(left,), device_id_type=pl.DeviceIdType.MESH)

        barrier = pltpu.get_barrier_semaphore()
        pl.semaphore_signal(barrier, inc=1, device_id=(left,),
                            device_id_type=pl.DeviceIdType.MESH)
        pl.semaphore_signal(barrier, inc=1, device_id=(right,),
                            device_id_type=pl.DeviceIdType.MESH)
        pl.semaphore_wait(barrier, 2)

        xallA[0] = x_ref[:mh, :]
        xallB[0] = x_ref[mh:, :]

        agA(0).start()
        agB(0).start()

        w1 = w1_ref[...]
        w2 = w2_ref[...]

        def partial_for(xb):
            h = jnp.dot(xb, w1, preferred_element_type=jnp.float32)
            h = h * jax.nn.sigmoid(h)
            return jnp.dot(h, w2, preferred_element_type=jnp.float32)

        pownA[...] = partial_for(xallA[0])
        pownB[...] = partial_for(xallB[0])

        def step(t, carry):
            slot = lax.rem(t, 2)
            prev_slot = lax.rem(t + 1, 2)

            agA(t).wait_recv()
            agB(t).wait_recv()

            @pl.when(t <= n_steps - 2)
            def _():
                agA(t + 1).start()
                agB(t + 1).start()

            agA(t).wait_send()
            agB(t).wait_send()

            pA = partial_for(xallA[pl.ds(t + 1, 1)].reshape(mh, d))
            pB = partial_for(xallB[pl.ds(t + 1, 1)].reshape(mh, d))

            @pl.when(t > 0)
            def _():
                rsA(t - 1).wait_recv()
                rsB(t - 1).wait_recv()

            prevA = rbufA[pl.ds(prev_slot, 1)].reshape(mh, n)
            prevB = rbufB[pl.ds(prev_slot, 1)].reshape(mh, n)
            accA = jnp.where(t > 0, pA + prevA, pA)
            accB = jnp.where(t > 0, pB + prevB, pB)

            @pl.when(jnp.logical_and(t >= 1, t <= n_steps - 2))
            def _():
                pl.semaphore_signal(creditA, inc=1, device_id=(left,),
                                    device_id_type=pl.DeviceIdType.MESH)
                pl.semaphore_signal(creditB, inc=1, device_id=(right,),
                                    device_id_type=pl.DeviceIdType.MESH)

            @pl.when(t >= 2)
            def _():
                rsA(t - 2).wait_send()
                rsB(t - 2).wait_send()

            sbufA[pl.ds(slot, 1)] = accA.reshape(1, mh, n)
            sbufB[pl.ds(slot, 1)] = accB.reshape(1, mh, n)

            @pl.when(t >= 2)
            def _():
                pl.semaphore_wait(creditA, 1)
                pl.semaphore_wait(creditB, 1)

            rsA(t).start()
            rsB(t).start()
            return carry

        lax.fori_loop(0, n_steps, step, 0)

        rsA(n_steps - 2).wait_send()
        rsB(n_steps - 2).wait_send()
        rsA(n_steps - 1).wait_send()
        rsB(n_steps - 1).wait_send()
        rsA(n_steps - 1).wait_recv()
        rsB(n_steps - 1).wait_recv()

        last_slot = (n_steps - 1) % 2
        out_ref[:mh, :] = rbufA[last_slot] + pownA[...]
        out_ref[mh:, :] = rbufB[last_slot] + pownB[...]

    me = lax.axis_index("i")
    nbrs = jnp.stack(
        [jnp.asarray(_LEFT_TBL)[me], jnp.asarray(_RIGHT_TBL)[me]]
    ).astype(jnp.int32)

    return pl.pallas_call(
        body,
        out_shape=jax.ShapeDtypeStruct((m, n), jnp.float32),
        in_specs=[pl.BlockSpec(memory_space=pltpu.SMEM)]
        + [pl.BlockSpec(memory_space=pltpu.VMEM)] * 3,
        out_specs=pl.BlockSpec(memory_space=pltpu.VMEM),
        scratch_shapes=[
            pltpu.VMEM((N_DEV, mh, d), jnp.float32),
            pltpu.VMEM((N_DEV, mh, d), jnp.float32),
            pltpu.VMEM((2, mh, n), jnp.float32),
            pltpu.VMEM((2, mh, n), jnp.float32),
            pltpu.VMEM((2, mh, n), jnp.float32),
            pltpu.VMEM((2, mh, n), jnp.float32),
            pltpu.VMEM((mh, n), jnp.float32),
            pltpu.VMEM((mh, n), jnp.float32),
            pltpu.SemaphoreType.DMA((n_steps,)),
            pltpu.SemaphoreType.DMA((n_steps,)),
            pltpu.SemaphoreType.DMA((n_steps,)),
            pltpu.SemaphoreType.DMA((n_steps,)),
            pltpu.SemaphoreType.DMA((n_steps,)),
            pltpu.SemaphoreType.DMA((n_steps,)),
            pltpu.SemaphoreType.DMA((n_steps,)),
            pltpu.SemaphoreType.DMA((n_steps,)),
            pltpu.SemaphoreType.REGULAR,
            pltpu.SemaphoreType.REGULAR,
        ],
        compiler_params=pltpu.CompilerParams(
            collective_id=0,
            vmem_limit_bytes=100 * 1024 * 1024,
        ),
    )(nbrs, x, W1, W2)


# device time: 370681 ns/iter; 2.4626x vs baseline; 1.0033x over previous
import jax
import jax.numpy as jnp
import numpy as np
from jax import lax
from jax.experimental import pallas as pl
from jax.experimental.pallas import tpu as pltpu

N_DEV = 32


def _ring_tables():
    path_yz = [(0, 0), (1, 0), (2, 0), (3, 0),
               (3, 1), (2, 1), (1, 1), (0, 1),
               (0, 2), (1, 2), (2, 2), (3, 2),
               (3, 3), (2, 3), (1, 3), (0, 3)]
    cycle = [(0, y, z) for (y, z) in path_yz] + \
            [(1, y, z) for (y, z) in reversed(path_yz)]

    def lid(x, y, z):
        return z * 8 + y * 2 + (x if y % 2 == 0 else 1 - x)

    ring = [lid(*c) for c in cycle]
    right = [0] * N_DEV
    left = [0] * N_DEV
    for r, lg in enumerate(ring):
        right[lg] = ring[(r + 1) % N_DEV]
        left[lg] = ring[(r - 1) % N_DEV]
    return np.array(right, np.int32), np.array(left, np.int32)


_RIGHT_TBL, _LEFT_TBL = _ring_tables()


def kernel(x, W1, W2):
    m, d = x.shape
    n = W2.shape[1]
    mh = m // 2
    n_steps = N_DEV - 1

    def body(nbrs_ref, x_ref, w1_ref, w2_ref, out_ref,
             xallA, xallB, sbufA, rbufA, sbufB, rbufB,
             pownA, pownB,
             agA0_send, agA0_recv, agA1_send, agA1_recv,
             agB0_send, agB0_recv, agB1_send, agB1_recv,
             rsA_send, rsA_recv, rsB_send, rsB_recv,
             creditA, creditB):
        left = nbrs_ref[0]
        right = nbrs_ref[1]
        mq = mh // 2

        def agA(s, c, snd, rcv):
            return pltpu.make_async_remote_copy(
                src_ref=xallA.at[s, pl.ds(c * mq, mq)],
                dst_ref=xallA.at[s + 1, pl.ds(c * mq, mq)],
                send_sem=snd.at[s], recv_sem=rcv.at[s],
                device_id=(right,), device_id_type=pl.DeviceIdType.MESH)

        def agA0(s):
            return agA(s, 0, agA0_send, agA0_recv)

        def agA1(s):
            return agA(s, 1, agA1_send, agA1_recv)

        def agB(s, c, snd, rcv):
            return pltpu.make_async_remote_copy(
                src_ref=xallB.at[s, pl.ds(c * mq, mq)],
                dst_ref=xallB.at[s + 1, pl.ds(c * mq, mq)],
                send_sem=snd.at[s], recv_sem=rcv.at[s],
                device_id=(left,), device_id_type=pl.DeviceIdType.MESH)

        def agB0(s):
            return agB(s, 0, agB0_send, agB0_recv)

        def agB1(s):
            return agB(s, 1, agB1_send, agB1_recv)

        def rsA(t):
            slot = lax.rem(t, 2)
            return pltpu.make_async_remote_copy(
                src_ref=sbufA.at[slot], dst_ref=rbufA.at[slot],
                send_sem=rsA_send.at[t], recv_sem=rsA_recv.at[t],
                device_id=(right,), device_id_type=pl.DeviceIdType.MESH)

        def rsB(t):
            slot = lax.rem(t, 2)
            return pltpu.make_async_remote_copy(
                src_ref=sbufB.at[slot], dst_ref=rbufB.at[slot],
                send_sem=rsB_send.at[t], recv_sem=rsB_recv.at[t],
                device_id=(left,), device_id_type=pl.DeviceIdType.MESH)

        barrier = pltpu.get_barrier_semaphore()
        pl.semaphore_signal(barrier, inc=1, device_id=(left,),
                            device_id_type=pl.DeviceIdType.MESH)
        pl.semaphore_signal(barrier, inc=1, device_id=(right,),
                            device_id_type=pl.DeviceIdType.MESH)
        pl.semaphore_wait(barrier, 2)

        xallA[0] = x_ref[:mh, :]
        xallB[0] = x_ref[mh:, :]

        agA0(0).start()
        agB0(0).start()
        agA1(0).start()
        agB1(0).start()

        w1 = w1_ref[...]
        w2 = w2_ref[...]

        def partial_for(xb):
            h = jnp.dot(xb, w1, preferred_element_type=jnp.float32)
            h = h * jax.nn.sigmoid(h)
            return jnp.dot(h, w2, preferred_element_type=jnp.float32)

        pownA[...] = partial_for(xallA[0])
        pownB[...] = partial_for(xallB[0])

        def step(t, carry):
            slot = lax.rem(t, 2)
            prev_slot = lax.rem(t + 1, 2)

            agA0(t).wait_recv()
            agB0(t).wait_recv()

            @pl.when(t <= n_steps - 2)
            def _():
                agA0(t + 1).start()
                agB0(t + 1).start()

            agA1(t).wait_recv()
            agB1(t).wait_recv()

            @pl.when(t <= n_steps - 2)
            def _():
                agA1(t + 1).start()
                agB1(t + 1).start()

            agA0(t).wait_send()
            agA1(t).wait_send()
            agB0(t).wait_send()
            agB1(t).wait_send()

            pA = partial_for(xallA[pl.ds(t + 1, 1)].reshape(mh, d))
            pB = partial_for(xallB[pl.ds(t + 1, 1)].reshape(mh, d))

            @pl.when(t > 0)
            def _():
                rsA(t - 1).wait_recv()
                rsB(t - 1).wait_recv()

            prevA = rbufA[pl.ds(prev_slot, 1)].reshape(mh, n)
            prevB = rbufB[pl.ds(prev_slot, 1)].reshape(mh, n)
            accA = jnp.where(t > 0, pA + prevA, pA)
            accB = jnp.where(t > 0, pB + prevB, pB)

            @pl.when(jnp.logical_and(t >= 1, t <= n_steps - 2))
            def _():
                pl.semaphore_signal(creditA, inc=1, device_id=(left,),
                                    device_id_type=pl.DeviceIdType.MESH)
                pl.semaphore_signal(creditB, inc=1, device_id=(right,),
                                    device_id_type=pl.DeviceIdType.MESH)

            @pl.when(t >= 2)
            def _():
                rsA(t - 2).wait_send()
                rsB(t - 2).wait_send()

            sbufA[pl.ds(slot, 1)] = accA.reshape(1, mh, n)
            sbufB[pl.ds(slot, 1)] = accB.reshape(1, mh, n)

            @pl.when(t >= 2)
            def _():
                pl.semaphore_wait(creditA, 1)
                pl.semaphore_wait(creditB, 1)

            rsA(t).start()
            rsB(t).start()
            return carry

        lax.fori_loop(0, n_steps, step, 0)

        rsA(n_steps - 2).wait_send()
        rsB(n_steps - 2).wait_send()
        rsA(n_steps - 1).wait_send()
        rsB(n_steps - 1).wait_send()
        rsA(n_steps - 1).wait_recv()
        rsB(n_steps - 1).wait_recv()

        last_slot = (n_steps - 1) % 2
        out_ref[:mh, :] = rbufA[last_slot] + pownA[...]
        out_ref[mh:, :] = rbufB[last_slot] + pownB[...]

    me = lax.axis_index("i")
    nbrs = jnp.stack(
        [jnp.asarray(_LEFT_TBL)[me], jnp.asarray(_RIGHT_TBL)[me]]
    ).astype(jnp.int32)

    return pl.pallas_call(
        body,
        out_shape=jax.ShapeDtypeStruct((m, n), jnp.float32),
        in_specs=[pl.BlockSpec(memory_space=pltpu.SMEM)]
        + [pl.BlockSpec(memory_space=pltpu.VMEM)] * 3,
        out_specs=pl.BlockSpec(memory_space=pltpu.VMEM),
        scratch_shapes=[
            pltpu.VMEM((N_DEV, mh, d), jnp.float32),
            pltpu.VMEM((N_DEV, mh, d), jnp.float32),
            pltpu.VMEM((2, mh, n), jnp.float32),
            pltpu.VMEM((2, mh, n), jnp.float32),
            pltpu.VMEM((2, mh, n), jnp.float32),
            pltpu.VMEM((2, mh, n), jnp.float32),
            pltpu.VMEM((mh, n), jnp.float32),
            pltpu.VMEM((mh, n), jnp.float32),
            pltpu.SemaphoreType.DMA((n_steps,)),
            pltpu.SemaphoreType.DMA((n_steps,)),
            pltpu.SemaphoreType.DMA((n_steps,)),
            pltpu.SemaphoreType.DMA((n_steps,)),
            pltpu.SemaphoreType.DMA((n_steps,)),
            pltpu.SemaphoreType.DMA((n_steps,)),
            pltpu.SemaphoreType.DMA((n_steps,)),
            pltpu.SemaphoreType.DMA((n_steps,)),
            pltpu.SemaphoreType.DMA((n_steps,)),
            pltpu.SemaphoreType.DMA((n_steps,)),
            pltpu.SemaphoreType.DMA((n_steps,)),
            pltpu.SemaphoreType.DMA((n_steps,)),
            pltpu.SemaphoreType.REGULAR,
            pltpu.SemaphoreType.REGULAR,
        ],
        compiler_params=pltpu.CompilerParams(
            collective_id=0,
            vmem_limit_bytes=100 * 1024 * 1024,
        ),
    )(nbrs, x, W1, W2)
